# baseline (speedup 1.0000x reference)
import jax
import jax.numpy as jnp
from jax import lax
from jax.experimental import pallas as pl

_EPS = 1e-05
_MAX_VALUE = 1.0 / (1.0 + _EPS)


def _tc_body(x_ref, o_ref):
    x = x_ref[...]
    col = lax.broadcasted_iota(jnp.int32, x.shape, 1)
    rm = jnp.max(jnp.where(col >= 1, x, jnp.float32(-jnp.inf)), axis=1,
                 keepdims=True)
    s = rm * jnp.float32(127.0)
    scale = s / (s + jnp.float32(_EPS))
    out = jnp.maximum(x - rm + jnp.float32(_MAX_VALUE), 0.0) * scale
    out0 = jnp.float32(_EPS) * (
        jnp.maximum(x + jnp.float32(_MAX_VALUE), 0.0) + 1.0)
    o_ref[...] = jnp.where(col == 0, out0, out)


def _tc_slice(X, row_start, rows, block_rows=512):
    L = X.shape[1]
    return pl.pallas_call(
        _tc_body,
        grid=(rows // block_rows,),
        in_specs=[pl.BlockSpec(
            (block_rows, L),
            lambda i, rs=row_start // block_rows: (rs + i, 0))],
        out_specs=pl.BlockSpec((block_rows, L), lambda i: (i, 0)),
        out_shape=jax.ShapeDtypeStruct((rows, L), X.dtype),
    )(X)


def kernel(X):
    N, L = X.shape
    return _tc_slice(X, 0, N)
